# baseline (speedup 1.0000x reference)
import jax
import jax.numpy as jnp
from jax import lax
from jax.experimental import pallas as pl
from jax.experimental.pallas import tpu as pltpu
from jax.experimental.pallas import tpu_sc as plsc

HIDDEN = 128
EPS = 1e-12
NC, NS, LANES = 2, 16, 16
NW = NC * NS
N_TOKENS = 1024 * 200
ROWS_PER_W = N_TOKENS // NW
CHUNK = 128
N_CHUNKS = ROWS_PER_W // CHUNK


def _rsqrt16(x):
    i = lax.bitcast_convert_type(x, jnp.int32)
    i = jnp.int32(0x5F3759DF) - lax.shift_right_arithmetic(i, jnp.int32(1))
    y = lax.bitcast_convert_type(i, jnp.float32)
    for _ in range(3):
        y = y * (1.5 - 0.5 * x * y * y)
    return y


def _body(ids_hbm, tt_hbm, word_hbm, type_hbm, gamma_hbm, beta_hbm, out_hbm,
          idx_v, tt_v, buf_v, type_v, gamma_v, beta_v, sem):
    wid = lax.axis_index("s") * NC + lax.axis_index("c")
    base = wid * ROWS_PER_W
    pltpu.sync_copy(type_hbm, type_v)
    pltpu.sync_copy(gamma_hbm, gamma_v)
    pltpu.sync_copy(beta_hbm, beta_v)
    lanes = lax.iota(jnp.int32, 16)

    def chunk(ci, _c):
        rbase = base + ci * CHUNK
        pltpu.sync_copy(ids_hbm.at[pl.ds(rbase, CHUNK)], idx_v)
        pltpu.sync_copy(tt_hbm.at[pl.ds(rbase, CHUNK)], tt_v)
        pltpu.async_copy(word_hbm.at[idx_v], buf_v, sem).wait()

        def group(g, _g):
            rows = g * LANES + lanes
            tt = plsc.load_gather(tt_v, [rows])

            def pass1(j, carry):
                s, ss = carry
                jj = jnp.full((LANES,), 0, jnp.int32) + j
                w = plsc.load_gather(buf_v, [rows, jj])
                t = plsc.load_gather(type_v, [tt, jj])
                x = w + t
                plsc.store_scatter(buf_v, [rows, jj], x)
                return (s + x, ss + x * x)

            zeros = jnp.zeros((LANES,), jnp.float32)
            s, ss = lax.fori_loop(0, HIDDEN, pass1, (zeros, zeros))
            mu = s * (1.0 / HIDDEN)
            var = ss * (1.0 / HIDDEN) - mu * mu
            rinv = _rsqrt16(var + EPS)

            def pass2(j, c2):
                jj = jnp.full((LANES,), 0, jnp.int32) + j
                x = plsc.load_gather(buf_v, [rows, jj])
                gsc = plsc.load_gather(gamma_v, [jj])
                bsc = plsc.load_gather(beta_v, [jj])
                y = (x - mu) * rinv * gsc + bsc
                plsc.store_scatter(buf_v, [rows, jj], y)
                return c2

            lax.fori_loop(0, HIDDEN, pass2, 0)
            return _g

        lax.fori_loop(0, CHUNK // LANES, group, 0)
        pltpu.sync_copy(buf_v, out_hbm.at[pl.ds(rbase, CHUNK)])
        return _c

    lax.fori_loop(0, N_CHUNKS, chunk, 0)


def kernel(input_ids, token_type_ids, word_emb, type_emb, gamma, beta):
    b, l = input_ids.shape
    ids = input_ids.reshape(-1).astype(jnp.int32)
    tts = token_type_ids.reshape(-1).astype(jnp.int32)
    run = pl.kernel(
        _body,
        out_type=jax.ShapeDtypeStruct((N_TOKENS, HIDDEN), jnp.float32),
        mesh=plsc.VectorSubcoreMesh(core_axis_name="c", subcore_axis_name="s"),
        scratch_types=[
            pltpu.VMEM((CHUNK,), jnp.int32),
            pltpu.VMEM((CHUNK,), jnp.int32),
            pltpu.VMEM((CHUNK, HIDDEN), jnp.float32),
            pltpu.VMEM((2, HIDDEN), jnp.float32),
            pltpu.VMEM((HIDDEN,), jnp.float32),
            pltpu.VMEM((HIDDEN,), jnp.float32),
            pltpu.SemaphoreType.DMA,
        ],
        compiler_params=pltpu.CompilerParams(needs_layout_passes=False),
    )
    out = run(ids, tts, word_emb, type_emb, gamma, beta)
    return out.reshape(b, l, HIDDEN)
